# Initial kernel scaffold; baseline (speedup 1.0000x reference)
import functools

import jax
import jax.numpy as jnp
from jax import lax
from jax.experimental import pallas as pl
from jax.experimental.pallas import tpu as pltpu
from jax.experimental.pallas import tpu_sc as plsc

N = 10000
E = 320000
D = 128

NC = 2
NS = 16
NW = NC * NS

BLK = 128
NBLK = -(-E // (NW * BLK))
PER_W = NBLK * BLK
E_PAD = PER_W * NW
N_ACC = N + 16
ZROWS = N_ACC // NS
OROWS = N // NS

_sc_mesh = plsc.VectorSubcoreMesh(core_axis_name="c", subcore_axis_name="s")


def _segsum_body(with_counts, t_hbm, src_hbm, dst_hbm, zf_hbm, zc_hbm,
                 ones_hbm, out_hbm, outc_hbm, src_v, dst_v, rows_v, ones_v,
                 acc_s, accc_s, sem):
    c = lax.axis_index("c")
    s = lax.axis_index("s")
    wid = c * NS + s
    pltpu.sync_copy(zf_hbm, acc_s.at[pl.ds(s * ZROWS, ZROWS)])
    if with_counts:
        pltpu.sync_copy(zc_hbm, accc_s.at[pl.ds(s * ZROWS, ZROWS)])
        pltpu.sync_copy(ones_hbm, ones_v)
    plsc.subcore_barrier()

    base = wid * PER_W

    @pl.loop(0, NBLK)
    def _(i):
        off = base + i * BLK
        pltpu.sync_copy(src_hbm.at[pl.ds(off, BLK)], src_v)
        pltpu.sync_copy(dst_hbm.at[pl.ds(off, BLK)], dst_v)
        pltpu.async_copy(t_hbm.at[src_v], rows_v, sem).wait()
        pltpu.sync_copy(rows_v, acc_s.at[dst_v], add=True)
        if with_counts:
            pltpu.sync_copy(ones_v, accc_s.at[dst_v], add=True)

    plsc.subcore_barrier()
    pltpu.sync_copy(acc_s.at[pl.ds(s * OROWS, OROWS)],
                    out_hbm.at[c, pl.ds(s * OROWS, OROWS)])
    if with_counts:
        pltpu.sync_copy(accc_s.at[pl.ds(s * OROWS, OROWS)],
                        outc_hbm.at[c, pl.ds(s * OROWS, OROWS)])


@functools.partial(
    pl.kernel,
    out_type=(jax.ShapeDtypeStruct((NC, N, D), jnp.float32),
              jax.ShapeDtypeStruct((NC, N, 16), jnp.float32)),
    mesh=_sc_mesh,
    scratch_types=[
        pltpu.VMEM((BLK,), jnp.int32),
        pltpu.VMEM((BLK,), jnp.int32),
        pltpu.VMEM((BLK, D), jnp.float32),
        pltpu.VMEM((BLK, 16), jnp.float32),
        pltpu.VMEM_SHARED((N_ACC, D), jnp.float32),
        pltpu.VMEM_SHARED((N_ACC, 16), jnp.float32),
        pltpu.SemaphoreType.DMA,
    ],
)
def _sc_segsum_counts(t_hbm, src_hbm, dst_hbm, zf_hbm, zc_hbm, ones_hbm,
                      out_hbm, outc_hbm, src_v, dst_v, rows_v, ones_v,
                      acc_s, accc_s, sem):
    _segsum_body(True, t_hbm, src_hbm, dst_hbm, zf_hbm, zc_hbm, ones_hbm,
                 out_hbm, outc_hbm, src_v, dst_v, rows_v, ones_v,
                 acc_s, accc_s, sem)


@functools.partial(
    pl.kernel,
    out_type=jax.ShapeDtypeStruct((NC, N, D), jnp.float32),
    mesh=_sc_mesh,
    scratch_types=[
        pltpu.VMEM((BLK,), jnp.int32),
        pltpu.VMEM((BLK,), jnp.int32),
        pltpu.VMEM((BLK, D), jnp.float32),
        pltpu.VMEM_SHARED((N_ACC, D), jnp.float32),
        pltpu.SemaphoreType.DMA,
    ],
)
def _sc_segsum(t_hbm, src_hbm, dst_hbm, zf_hbm, out_hbm, src_v, dst_v,
               rows_v, acc_s, sem):
    _segsum_body(False, t_hbm, src_hbm, dst_hbm, zf_hbm, None, None,
                 out_hbm, None, src_v, dst_v, rows_v, None, acc_s, None, sem)


_PREC = lax.Precision.HIGHEST


def _pre_body(x_ref, wl_ref, wr_ref, b_ref, t_ref, r_ref):
    x = x_ref[...]
    t_ref[...] = jnp.dot(x, wl_ref[...], preferred_element_type=jnp.float32,
                         precision=_PREC)
    r_ref[...] = jnp.dot(x, wr_ref[...], preferred_element_type=jnp.float32,
                         precision=_PREC) + b_ref[...]


def _mid_body(p_ref, cp_ref, r0_ref, g_ref, bt_ref, wl1_ref, wr1_ref, b1_ref,
              t1_ref, r1_ref):
    cnt = cp_ref[0, :, :1] + cp_ref[1, :, :1]
    inv = 1.0 / jnp.maximum(cnt, 1.0)
    h = (p_ref[0] + p_ref[1]) * inv + r0_ref[...]
    mu = jnp.mean(h, axis=0, keepdims=True)
    var = jnp.mean((h - mu) * (h - mu), axis=0, keepdims=True)
    hn = (h - mu) * lax.rsqrt(var + 1e-5) * g_ref[...] + bt_ref[...]
    h2 = jnp.maximum(hn, 0.0)
    t1_ref[...] = jnp.dot(h2, wl1_ref[...], preferred_element_type=jnp.float32,
                          precision=_PREC)
    r1_ref[...] = jnp.dot(h2, wr1_ref[...], preferred_element_type=jnp.float32,
                          precision=_PREC) + b1_ref[...]


def _fin_body(q_ref, cp_ref, r1_ref, o_ref):
    cnt = cp_ref[0, :, :1] + cp_ref[1, :, :1]
    inv = 1.0 / jnp.maximum(cnt, 1.0)
    o_ref[...] = (q_ref[0] + q_ref[1]) * inv + r1_ref[...]


def kernel(x, edge_index, W_l0, b_l0, W_r0, gamma, beta, W_l1, b_l1, W_r1):
    src = edge_index[0]
    dst = edge_index[1]
    pad = E_PAD - E
    src_p = jnp.concatenate([src, jnp.zeros((pad,), jnp.int32)])
    dst_p = jnp.concatenate([dst, jnp.full((pad,), N, jnp.int32)])

    zf = jnp.zeros((ZROWS, D), jnp.float32)
    zc = jnp.zeros((ZROWS, 16), jnp.float32)
    ones = jnp.ones((BLK, 16), jnp.float32)

    f32 = jnp.float32
    t0, r0 = pl.pallas_call(
        _pre_body,
        out_shape=(jax.ShapeDtypeStruct((N, D), f32),
                   jax.ShapeDtypeStruct((N, D), f32)),
    )(x, W_l0, W_r0, b_l0.reshape(1, D))

    p0, cp = _sc_segsum_counts(t0, src_p, dst_p, zf, zc, ones)

    t1, r1 = pl.pallas_call(
        _mid_body,
        out_shape=(jax.ShapeDtypeStruct((N, D), f32),
                   jax.ShapeDtypeStruct((N, D), f32)),
    )(p0, cp, r0, gamma.reshape(1, D), beta.reshape(1, D), W_l1, W_r1,
      b_l1.reshape(1, D))

    q1 = _sc_segsum(t1, src_p, dst_p, zf)

    out = pl.pallas_call(
        _fin_body,
        out_shape=jax.ShapeDtypeStruct((N, D), f32),
    )(q1, cp, r1)
    return out


# speedup vs baseline: 3.9709x; 3.9709x over previous
import functools

import jax
import jax.numpy as jnp
from jax import lax
from jax.experimental import pallas as pl
from jax.experimental.pallas import tpu as pltpu
from jax.experimental.pallas import tpu_sc as plsc

N = 10000
E = 320000
D = 128

NC = 2
NS = 16
NW = NC * NS
CW = 128

BLK = 128
NBLK = -(-E // (NW * BLK))
PER_W = NBLK * BLK
E_PAD = PER_W * NW
N_PAD = 10240
ZROWS = N_PAD // NS

_sc_mesh = plsc.VectorSubcoreMesh(core_axis_name="c", subcore_axis_name="s")


@functools.partial(
    pl.kernel,
    out_type=jax.ShapeDtypeStruct((NC, N_PAD, D), jnp.float32),
    mesh=_sc_mesh,
    scratch_types=[
        pltpu.VMEM((BLK,), jnp.int32),
        pltpu.VMEM((BLK,), jnp.int32),
        pltpu.VMEM((BLK, D), jnp.float32),
        pltpu.VMEM_SHARED((N_PAD, D), jnp.float32),
        pltpu.SemaphoreType.DMA,
    ],
)
def _sc_segsum(t_hbm, src_hbm, dst_hbm, zf_hbm, out_hbm, src_v, dst_v,
               rows_v, acc_s, sem):
    c = lax.axis_index("c")
    s = lax.axis_index("s")
    wid = c * NS + s
    pltpu.sync_copy(zf_hbm, acc_s.at[pl.ds(s * ZROWS, ZROWS)])
    plsc.subcore_barrier()

    base = wid * PER_W

    @pl.loop(0, NBLK)
    def _(i):
        off = base + i * BLK
        pltpu.sync_copy(src_hbm.at[pl.ds(off, BLK)], src_v)
        pltpu.sync_copy(dst_hbm.at[pl.ds(off, BLK)], dst_v)
        pltpu.async_copy(t_hbm.at[src_v], rows_v, sem).wait()
        pltpu.sync_copy(rows_v, acc_s.at[dst_v], add=True)

    plsc.subcore_barrier()
    pltpu.sync_copy(acc_s.at[pl.ds(s * ZROWS, ZROWS)],
                    out_hbm.at[c, pl.ds(s * ZROWS, ZROWS)])


@functools.partial(
    pl.kernel,
    out_type=jax.ShapeDtypeStruct((NC, N_PAD, CW), jnp.float32),
    mesh=_sc_mesh,
    scratch_types=[
        pltpu.VMEM((BLK,), jnp.int32),
        pltpu.VMEM((BLK, CW), jnp.float32),
        pltpu.VMEM_SHARED((N_PAD, CW), jnp.float32),
    ],
)
def _sc_counts(dst_hbm, zc_hbm, ones_hbm, outc_hbm, dst_v, ones_v, accc_s):
    c = lax.axis_index("c")
    s = lax.axis_index("s")
    wid = c * NS + s
    pltpu.sync_copy(zc_hbm, accc_s.at[pl.ds(s * ZROWS, ZROWS)])
    pltpu.sync_copy(ones_hbm, ones_v)
    plsc.subcore_barrier()

    base = wid * PER_W

    @pl.loop(0, NBLK)
    def _(i):
        off = base + i * BLK
        pltpu.sync_copy(dst_hbm.at[pl.ds(off, BLK)], dst_v)
        pltpu.sync_copy(ones_v, accc_s.at[dst_v], add=True)

    plsc.subcore_barrier()
    pltpu.sync_copy(accc_s.at[pl.ds(s * ZROWS, ZROWS)],
                    outc_hbm.at[c, pl.ds(s * ZROWS, ZROWS)])


_PREC = lax.Precision.HIGHEST


def _pre_body(x_ref, wl_ref, wr_ref, b_ref, t_ref, r_ref):
    x = x_ref[...]
    t_ref[...] = jnp.dot(x, wl_ref[...], preferred_element_type=jnp.float32,
                         precision=_PREC)
    r_ref[...] = jnp.dot(x, wr_ref[...], preferred_element_type=jnp.float32,
                         precision=_PREC) + b_ref[...]


def _mid1_body(p_ref, cp_ref, r0_ref, h_ref, mu_ref, var_ref):
    cnt = cp_ref[0, :N, :1] + cp_ref[1, :N, :1]
    inv = 1.0 / jnp.maximum(cnt, 1.0)
    h = (p_ref[0, :N] + p_ref[1, :N]) * inv + r0_ref[...]
    mu = jnp.mean(h, axis=0, keepdims=True)
    var = jnp.mean((h - mu) * (h - mu), axis=0, keepdims=True)
    h_ref[...] = h
    mu_ref[...] = jnp.broadcast_to(mu, (8, D))
    var_ref[...] = jnp.broadcast_to(var, (8, D))


def _mid2_body(h_ref, mu_ref, var_ref, g_ref, bt_ref, wl1_ref, wr1_ref,
               b1_ref, t1_ref, r1_ref):
    mu = mu_ref[:1, :]
    var = var_ref[:1, :]
    hn = (h_ref[...] - mu) * lax.rsqrt(var + 1e-5) * g_ref[...] + bt_ref[...]
    h2 = jnp.maximum(hn, 0.0)
    t1_ref[...] = jnp.dot(h2, wl1_ref[...], preferred_element_type=jnp.float32,
                          precision=_PREC)
    r1_ref[...] = jnp.dot(h2, wr1_ref[...], preferred_element_type=jnp.float32,
                          precision=_PREC) + b1_ref[...]


def _fin_body(q_ref, cp_ref, r1_ref, o_ref):
    cnt = cp_ref[0, :N, :1] + cp_ref[1, :N, :1]
    inv = 1.0 / jnp.maximum(cnt, 1.0)
    o_ref[...] = (q_ref[0, :N] + q_ref[1, :N]) * inv + r1_ref[...]


def kernel(x, edge_index, W_l0, b_l0, W_r0, gamma, beta, W_l1, b_l1, W_r1):
    src = edge_index[0]
    dst = edge_index[1]
    pad = E_PAD - E
    src_p = jnp.concatenate([src, jnp.zeros((pad,), jnp.int32)])
    dst_p = jnp.concatenate([dst, jnp.full((pad,), N, jnp.int32)])

    zf = jnp.zeros((ZROWS, D), jnp.float32)
    zc = jnp.zeros((ZROWS, CW), jnp.float32)
    ones = jnp.ones((BLK, CW), jnp.float32)

    f32 = jnp.float32
    t0, r0 = pl.pallas_call(
        _pre_body,
        out_shape=(jax.ShapeDtypeStruct((N, D), f32),
                   jax.ShapeDtypeStruct((N, D), f32)),
    )(x, W_l0, W_r0, b_l0.reshape(1, D))

    cp = _sc_counts(dst_p, zc, ones)
    p0 = _sc_segsum(t0, src_p, dst_p, zf)

    h, mu, var = pl.pallas_call(
        _mid1_body,
        out_shape=(jax.ShapeDtypeStruct((N, D), f32),
                   jax.ShapeDtypeStruct((8, D), f32),
                   jax.ShapeDtypeStruct((8, D), f32)),
    )(p0, cp, r0)

    t1, r1 = pl.pallas_call(
        _mid2_body,
        out_shape=(jax.ShapeDtypeStruct((N, D), f32),
                   jax.ShapeDtypeStruct((N, D), f32)),
    )(h, mu, var, gamma.reshape(1, D), beta.reshape(1, D), W_l1, W_r1,
      b_l1.reshape(1, D))

    q1 = _sc_segsum(t1, src_p, dst_p, zf)

    out = pl.pallas_call(
        _fin_body,
        out_shape=jax.ShapeDtypeStruct((N, D), f32),
    )(q1, cp, r1)
    return out
